# baseline (speedup 1.0000x reference)
import functools

import jax
import jax.numpy as jnp
from jax import lax
from jax.experimental import pallas as pl
from jax.experimental.pallas import tpu as pltpu
from jax.experimental.pallas import tpu_sc as plsc

NC = 2
NS = 16
NW = NC * NS
EB = 128
DEG_W = 16


def _sc_segment_sum(feat, srcc, dstw, n_acc):
  _, D = feat.shape
  _, PB, _ = dstw.shape
  HB = PB // 2
  rpt = n_acc // NS
  mesh = plsc.VectorSubcoreMesh(
      core_axis_name="c", subcore_axis_name="s", num_cores=NC, num_subcores=NS)

  @functools.partial(
      pl.kernel,
      out_type=jax.ShapeDtypeStruct((NC, n_acc, D), jnp.float32),
      mesh=mesh,
      scratch_types=[
          pltpu.VMEM((HB + 1, EB), jnp.int32),
          pltpu.VMEM((PB, EB), jnp.int32),
          pltpu.VMEM((EB, D), jnp.float32),
          pltpu.VMEM((EB, D), jnp.float32),
          pltpu.VMEM_SHARED((n_acc, D), jnp.float32),
          pltpu.SemaphoreType.DMA,
          pltpu.SemaphoreType.DMA,
      ],
  )
  def seg(feat_hbm, src_hbm, dst_hbm, out_hbm,
          srcv, dstv, rows0, rows1, acc, sem0, sem1):
    c = lax.axis_index("c")
    s = lax.axis_index("s")
    wid = s * NC + c
    pltpu.sync_copy(dst_hbm.at[wid], dstv)

    def zfill(i, carry):
      r = i // (D // 16)
      col = (i % (D // 16)) * 16
      rows0[r, pl.ds(col, 16)] = jnp.zeros((16,), jnp.float32)
      return carry
    lax.fori_loop(0, 16 * (D // 16), zfill, 0)

    base = s * rpt

    def zblock(i, carry):
      pltpu.sync_copy(rows0.at[pl.ds(0, 16)], acc.at[pl.ds(base + i * 16, 16)])
      return carry
    lax.fori_loop(0, rpt // 16, zblock, 0)
    plsc.subcore_barrier()

    def start_g(j, buf, sem):
      pltpu.async_copy(feat_hbm.at[srcv.at[j]], buf, sem)

    def wait_g(buf, sem):
      pltpu.make_async_copy(feat_hbm.at[srcv.at[0]], buf, sem).wait()

    def half(h):
      pltpu.sync_copy(src_hbm.at[wid * 2 + h], srcv)
      start_g(0, rows0, sem0)

      def step(k, carry):
        j0 = 2 * k
        wait_g(rows0, sem0)
        start_g(j0 + 1, rows1, sem1)
        pltpu.sync_copy(rows0, acc.at[dstv.at[h * HB + j0]], add=True)
        wait_g(rows1, sem1)
        start_g(j0 + 2, rows0, sem0)
        pltpu.sync_copy(rows1, acc.at[dstv.at[h * HB + j0 + 1]], add=True)
        return carry
      lax.fori_loop(0, HB // 2, step, 0)
      wait_g(rows0, sem0)

    half(0)
    half(1)
    plsc.subcore_barrier()

    pltpu.sync_copy(acc.at[pl.ds(base, rpt)], out_hbm.at[c, pl.ds(base, rpt)])

  return seg(feat, srcc, dstw)


def _sc_degree(dstw, n_acc):
  _, PB, _ = dstw.shape
  D = DEG_W
  rpt = n_acc // NS
  mesh = plsc.VectorSubcoreMesh(
      core_axis_name="c", subcore_axis_name="s", num_cores=NC, num_subcores=NS)

  @functools.partial(
      pl.kernel,
      out_type=jax.ShapeDtypeStruct((NC, n_acc, D), jnp.float32),
      mesh=mesh,
      scratch_types=[
          pltpu.VMEM((PB, EB), jnp.int32),
          pltpu.VMEM((EB, D), jnp.float32),
          pltpu.VMEM((16, D), jnp.float32),
          pltpu.VMEM_SHARED((n_acc, D), jnp.float32),
      ],
  )
  def deg(dst_hbm, out_hbm, dstv, onesv, zv, acc):
    c = lax.axis_index("c")
    s = lax.axis_index("s")
    wid = s * NC + c
    pltpu.sync_copy(dst_hbm.at[wid], dstv)

    def ofill(i, carry):
      onesv[i, pl.ds(0, 16)] = jnp.ones((16,), jnp.float32)
      return carry
    lax.fori_loop(0, EB, ofill, 0)

    def zfill(i, carry):
      zv[i, pl.ds(0, 16)] = jnp.zeros((16,), jnp.float32)
      return carry
    lax.fori_loop(0, 16, zfill, 0)

    base = s * rpt

    def zblock(i, carry):
      pltpu.sync_copy(zv, acc.at[pl.ds(base + i * 16, 16)])
      return carry
    lax.fori_loop(0, rpt // 16, zblock, 0)
    plsc.subcore_barrier()

    def step(j, carry):
      pltpu.sync_copy(onesv, acc.at[dstv.at[j]], add=True)
      return carry
    lax.fori_loop(0, PB, step, 0)
    plsc.subcore_barrier()

    pltpu.sync_copy(acc.at[pl.ds(base, rpt)], out_hbm.at[c, pl.ds(base, rpt)])

  return deg(dstw)


def _tc_dinv_scale(deg0, deg1, x, R):
  N, D = x.shape

  def body(d0, d1, xr, dinv_ref, xs_ref):
    dinv = lax.rsqrt(d0[...] + d1[...] + 1.0)
    dinv_ref[...] = dinv
    xs_ref[...] = xr[...] * dinv

  return pl.pallas_call(
      body,
      grid=(N // R,),
      in_specs=[
          pl.BlockSpec((R, 1), lambda i: (i, 0)),
          pl.BlockSpec((R, 1), lambda i: (i, 0)),
          pl.BlockSpec((R, D), lambda i: (i, 0)),
      ],
      out_specs=[
          pl.BlockSpec((R, 1), lambda i: (i, 0)),
          pl.BlockSpec((R, D), lambda i: (i, 0)),
      ],
      out_shape=[
          jax.ShapeDtypeStruct((N, 1), jnp.float32),
          jax.ShapeDtypeStruct((N, D), jnp.float32),
      ],
  )(deg0, deg1, x)


def _tc_agg_matmul_stats(a0, a1, xs, dinv, W, b, R):
  N, D = xs.shape
  DO = W.shape[1]

  def body(a0r, a1r, xsr, dinvr, Wr, br, h_ref, s_ref, q_ref):
    agg = (a0r[...] + a1r[...] + xsr[...]) * dinvr[...]
    h = jnp.dot(agg, Wr[...], preferred_element_type=jnp.float32) + br[...]
    h_ref[...] = h

    @pl.when(pl.program_id(0) == 0)
    def _():
      s_ref[...] = jnp.zeros_like(s_ref)
      q_ref[...] = jnp.zeros_like(q_ref)

    s_ref[...] += jnp.sum(h, axis=0, keepdims=True)
    q_ref[...] += jnp.sum(h * h, axis=0, keepdims=True)

  return pl.pallas_call(
      body,
      grid=(N // R,),
      in_specs=[
          pl.BlockSpec((R, D), lambda i: (i, 0)),
          pl.BlockSpec((R, D), lambda i: (i, 0)),
          pl.BlockSpec((R, D), lambda i: (i, 0)),
          pl.BlockSpec((R, 1), lambda i: (i, 0)),
          pl.BlockSpec((D, DO), lambda i: (0, 0)),
          pl.BlockSpec((1, DO), lambda i: (0, 0)),
      ],
      out_specs=[
          pl.BlockSpec((R, DO), lambda i: (i, 0)),
          pl.BlockSpec((1, DO), lambda i: (0, 0)),
          pl.BlockSpec((1, DO), lambda i: (0, 0)),
      ],
      out_shape=[
          jax.ShapeDtypeStruct((N, DO), jnp.float32),
          jax.ShapeDtypeStruct((1, DO), jnp.float32),
          jax.ShapeDtypeStruct((1, DO), jnp.float32),
      ],
  )(a0, a1, xs, dinv, W, b)


def _tc_bn_relu_matmul_scale(h, s, q, gamma, beta, W, dinv, R):
  N, D = h.shape
  DO = W.shape[1]
  inv_n = 1.0 / N

  def body(hr, sr, qr, gr, br, Wr, dinvr, gs_ref):
    mean = sr[...] * inv_n
    var = qr[...] * inv_n - mean * mean
    istd = lax.rsqrt(var + 1e-5)
    hn = (hr[...] - mean) * istd * gr[...] + br[...]
    a = jnp.maximum(hn, 0.0)
    g = jnp.dot(a, Wr[...], preferred_element_type=jnp.float32)
    gs_ref[...] = g * dinvr[...]

  return pl.pallas_call(
      body,
      grid=(N // R,),
      in_specs=[
          pl.BlockSpec((R, D), lambda i: (i, 0)),
          pl.BlockSpec((1, D), lambda i: (0, 0)),
          pl.BlockSpec((1, D), lambda i: (0, 0)),
          pl.BlockSpec((1, D), lambda i: (0, 0)),
          pl.BlockSpec((1, D), lambda i: (0, 0)),
          pl.BlockSpec((D, DO), lambda i: (0, 0)),
          pl.BlockSpec((R, 1), lambda i: (i, 0)),
      ],
      out_specs=pl.BlockSpec((R, DO), lambda i: (i, 0)),
      out_shape=jax.ShapeDtypeStruct((N, DO), jnp.float32),
  )(h, s, q, gamma, beta, W, dinv)


def _tc_agg_bias_stats(a0, a1, gs, dinv, b, R):
  N, D = gs.shape

  def body(a0r, a1r, gsr, dinvr, br, o_ref, s_ref, q_ref):
    op = (a0r[...] + a1r[...] + gsr[...]) * dinvr[...] + br[...]
    o_ref[...] = op

    @pl.when(pl.program_id(0) == 0)
    def _():
      s_ref[...] = jnp.zeros_like(s_ref)
      q_ref[...] = jnp.zeros_like(q_ref)

    s_ref[...] += jnp.sum(op, axis=0, keepdims=True)
    q_ref[...] += jnp.sum(op * op, axis=0, keepdims=True)

  return pl.pallas_call(
      body,
      grid=(N // R,),
      in_specs=[
          pl.BlockSpec((R, D), lambda i: (i, 0)),
          pl.BlockSpec((R, D), lambda i: (i, 0)),
          pl.BlockSpec((R, D), lambda i: (i, 0)),
          pl.BlockSpec((R, 1), lambda i: (i, 0)),
          pl.BlockSpec((1, D), lambda i: (0, 0)),
      ],
      out_specs=[
          pl.BlockSpec((R, D), lambda i: (i, 0)),
          pl.BlockSpec((1, D), lambda i: (0, 0)),
          pl.BlockSpec((1, D), lambda i: (0, 0)),
      ],
      out_shape=[
          jax.ShapeDtypeStruct((N, D), jnp.float32),
          jax.ShapeDtypeStruct((1, D), jnp.float32),
          jax.ShapeDtypeStruct((1, D), jnp.float32),
      ],
  )(a0, a1, gs, dinv, b)


def _tc_bn_apply(op, s, q, gamma, beta, R):
  N, D = op.shape
  inv_n = 1.0 / N

  def body(opr, sr, qr, gr, br, out_ref):
    mean = sr[...] * inv_n
    var = qr[...] * inv_n - mean * mean
    istd = lax.rsqrt(var + 1e-5)
    out_ref[...] = (opr[...] - mean) * istd * gr[...] + br[...]

  return pl.pallas_call(
      body,
      grid=(N // R,),
      in_specs=[
          pl.BlockSpec((R, D), lambda i: (i, 0)),
          pl.BlockSpec((1, D), lambda i: (0, 0)),
          pl.BlockSpec((1, D), lambda i: (0, 0)),
          pl.BlockSpec((1, D), lambda i: (0, 0)),
          pl.BlockSpec((1, D), lambda i: (0, 0)),
      ],
      out_specs=pl.BlockSpec((R, D), lambda i: (i, 0)),
      out_shape=jax.ShapeDtypeStruct((N, D), jnp.float32),
  )(op, s, q, gamma, beta)


def kernel(x, edge_index, W1, b1, gamma1, beta1, W2, b2, gamma2, beta2):
  N, _ = x.shape
  E = edge_index.shape[1]
  R = 2000

  n_acc = ((N + NS * 16 - 1) // (NS * 16)) * (NS * 16)

  src = edge_index[0].astype(jnp.int32)
  dst = edge_index[1].astype(jnp.int32)
  pb = (E + NW * EB - 1) // (NW * EB)
  pb = pb + (pb % 2)
  e_pad = NW * pb * EB
  pad = e_pad - E
  src_p = jnp.concatenate([src, jnp.zeros((pad,), jnp.int32)])
  dst_p = jnp.concatenate([dst, jnp.full((pad,), N, jnp.int32)])
  srcw = src_p.reshape(NW, pb, EB)
  dstw = dst_p.reshape(NW, pb, EB)

  hb = pb // 2
  dummy = jnp.zeros((NW, 2, 1, EB), jnp.int32)
  srcc = jnp.concatenate([srcw.reshape(NW, 2, hb, EB), dummy], axis=2)
  srcc = srcc.reshape(NW * 2, hb + 1, EB)

  degp = _sc_degree(dstw, n_acc)
  deg0 = degp[0, :N, 0:1]
  deg1 = degp[1, :N, 0:1]

  dinv, xs = _tc_dinv_scale(deg0, deg1, x, R)

  accA = _sc_segment_sum(xs, srcc, dstw, n_acc)
  h, s1, q1 = _tc_agg_matmul_stats(
      accA[0, :N], accA[1, :N], xs, dinv, W1, b1.reshape(1, -1), R)

  gs = _tc_bn_relu_matmul_scale(
      h, s1, q1, gamma1.reshape(1, -1), beta1.reshape(1, -1), W2, dinv, R)

  accB = _sc_segment_sum(gs, srcc, dstw, n_acc)
  op, s2, q2 = _tc_agg_bias_stats(
      accB[0, :N], accB[1, :N], gs, dinv, b2.reshape(1, -1), R)

  return _tc_bn_apply(op, s2, q2, gamma2.reshape(1, -1), beta2.reshape(1, -1), R)


# speedup vs baseline: 2.1501x; 2.1501x over previous
import functools

import jax
import jax.numpy as jnp
from jax import lax
from jax.experimental import pallas as pl
from jax.experimental.pallas import tpu as pltpu
from jax.experimental.pallas import tpu_sc as plsc

NC = 2
NS = 16
NW = NC * NS
EB = 128
DEG_W = 16


def _sc_segment_sum(feat, srcw, dstw, n_acc):
  _, D = feat.shape
  _, PB, _ = srcw.shape
  rpt = n_acc // NS
  mesh = plsc.VectorSubcoreMesh(
      core_axis_name="c", subcore_axis_name="s", num_cores=NC, num_subcores=NS)

  @functools.partial(
      pl.kernel,
      out_type=jax.ShapeDtypeStruct((NC, n_acc, D), jnp.float32),
      mesh=mesh,
      scratch_types=[
          pltpu.VMEM((PB, EB), jnp.int32),
          pltpu.VMEM((PB, EB), jnp.int32),
          pltpu.VMEM((EB, D), jnp.float32),
          pltpu.VMEM((16, D), jnp.float32),
          pltpu.VMEM_SHARED((n_acc, D), jnp.float32),
          pltpu.SemaphoreType.DMA,
      ],
  )
  def seg(feat_hbm, src_hbm, dst_hbm, out_hbm, srcv, dstv, rowsv, zv, acc, sem):
    c = lax.axis_index("c")
    s = lax.axis_index("s")
    wid = s * NC + c
    pltpu.sync_copy(src_hbm.at[wid], srcv)
    pltpu.sync_copy(dst_hbm.at[wid], dstv)

    def zfill(i, carry):
      r = i // (D // 16)
      col = (i % (D // 16)) * 16
      zv[r, pl.ds(col, 16)] = jnp.zeros((16,), jnp.float32)
      return carry
    lax.fori_loop(0, 16 * (D // 16), zfill, 0)

    base = s * rpt

    def zblock(i, carry):
      pltpu.sync_copy(zv, acc.at[pl.ds(base + i * 16, 16)])
      return carry
    lax.fori_loop(0, rpt // 16, zblock, 0)
    plsc.subcore_barrier()

    def step(j, carry):
      pltpu.async_copy(feat_hbm.at[srcv.at[j]], rowsv, sem).wait()
      pltpu.sync_copy(rowsv, acc.at[dstv.at[j]], add=True)
      return carry
    lax.fori_loop(0, PB, step, 0)
    plsc.subcore_barrier()

    pltpu.sync_copy(acc.at[pl.ds(base, rpt)], out_hbm.at[c, pl.ds(base, rpt)])

  return seg(feat, srcw, dstw)


def _sc_degree(dstw, n_acc):
  _, PB, _ = dstw.shape
  D = DEG_W
  rpt = n_acc // NS
  mesh = plsc.VectorSubcoreMesh(
      core_axis_name="c", subcore_axis_name="s", num_cores=NC, num_subcores=NS)

  @functools.partial(
      pl.kernel,
      out_type=jax.ShapeDtypeStruct((NC, n_acc, D), jnp.float32),
      mesh=mesh,
      scratch_types=[
          pltpu.VMEM((PB, EB), jnp.int32),
          pltpu.VMEM((EB, D), jnp.float32),
          pltpu.VMEM((16, D), jnp.float32),
          pltpu.VMEM_SHARED((n_acc, D), jnp.float32),
      ],
  )
  def deg(dst_hbm, out_hbm, dstv, onesv, zv, acc):
    c = lax.axis_index("c")
    s = lax.axis_index("s")
    wid = s * NC + c
    pltpu.sync_copy(dst_hbm.at[wid], dstv)

    def ofill(i, carry):
      onesv[i, pl.ds(0, 16)] = jnp.ones((16,), jnp.float32)
      return carry
    lax.fori_loop(0, EB, ofill, 0)

    def zfill(i, carry):
      zv[i, pl.ds(0, 16)] = jnp.zeros((16,), jnp.float32)
      return carry
    lax.fori_loop(0, 16, zfill, 0)

    base = s * rpt

    def zblock(i, carry):
      pltpu.sync_copy(zv, acc.at[pl.ds(base + i * 16, 16)])
      return carry
    lax.fori_loop(0, rpt // 16, zblock, 0)
    plsc.subcore_barrier()

    def step(j, carry):
      pltpu.sync_copy(onesv, acc.at[dstv.at[j]], add=True)
      return carry
    lax.fori_loop(0, PB, step, 0)
    plsc.subcore_barrier()

    pltpu.sync_copy(acc.at[pl.ds(base, rpt)], out_hbm.at[c, pl.ds(base, rpt)])

  return deg(dstw)


def _tc_dinv_scale(deg0, deg1, x, R):
  N, D = x.shape

  def body(d0, d1, xr, dinv_ref, xs_ref):
    dinv = lax.rsqrt(d0[...] + d1[...] + 1.0)
    dinv_ref[...] = dinv
    xs_ref[...] = xr[...] * dinv

  return pl.pallas_call(
      body,
      grid=(N // R,),
      in_specs=[
          pl.BlockSpec((R, 1), lambda i: (i, 0)),
          pl.BlockSpec((R, 1), lambda i: (i, 0)),
          pl.BlockSpec((R, D), lambda i: (i, 0)),
      ],
      out_specs=[
          pl.BlockSpec((R, 1), lambda i: (i, 0)),
          pl.BlockSpec((R, D), lambda i: (i, 0)),
      ],
      out_shape=[
          jax.ShapeDtypeStruct((N, 1), jnp.float32),
          jax.ShapeDtypeStruct((N, D), jnp.float32),
      ],
  )(deg0, deg1, x)


def _tc_agg_matmul_stats(a0, a1, xs, dinv, W, b, R):
  N, D = xs.shape
  DO = W.shape[1]

  def body(a0r, a1r, xsr, dinvr, Wr, br, h_ref, s_ref, q_ref):
    agg = (a0r[...] + a1r[...] + xsr[...]) * dinvr[...]
    h = jnp.dot(agg, Wr[...], preferred_element_type=jnp.float32) + br[...]
    h_ref[...] = h

    @pl.when(pl.program_id(0) == 0)
    def _():
      s_ref[...] = jnp.zeros_like(s_ref)
      q_ref[...] = jnp.zeros_like(q_ref)

    s_ref[...] += jnp.sum(h, axis=0, keepdims=True)
    q_ref[...] += jnp.sum(h * h, axis=0, keepdims=True)

  return pl.pallas_call(
      body,
      grid=(N // R,),
      in_specs=[
          pl.BlockSpec((R, D), lambda i: (i, 0)),
          pl.BlockSpec((R, D), lambda i: (i, 0)),
          pl.BlockSpec((R, D), lambda i: (i, 0)),
          pl.BlockSpec((R, 1), lambda i: (i, 0)),
          pl.BlockSpec((D, DO), lambda i: (0, 0)),
          pl.BlockSpec((1, DO), lambda i: (0, 0)),
      ],
      out_specs=[
          pl.BlockSpec((R, DO), lambda i: (i, 0)),
          pl.BlockSpec((1, DO), lambda i: (0, 0)),
          pl.BlockSpec((1, DO), lambda i: (0, 0)),
      ],
      out_shape=[
          jax.ShapeDtypeStruct((N, DO), jnp.float32),
          jax.ShapeDtypeStruct((1, DO), jnp.float32),
          jax.ShapeDtypeStruct((1, DO), jnp.float32),
      ],
  )(a0, a1, xs, dinv, W, b)


def _tc_bn_relu_matmul_scale(h, s, q, gamma, beta, W, dinv, R):
  N, D = h.shape
  DO = W.shape[1]
  inv_n = 1.0 / N

  def body(hr, sr, qr, gr, br, Wr, dinvr, gs_ref):
    mean = sr[...] * inv_n
    var = qr[...] * inv_n - mean * mean
    istd = lax.rsqrt(var + 1e-5)
    hn = (hr[...] - mean) * istd * gr[...] + br[...]
    a = jnp.maximum(hn, 0.0)
    g = jnp.dot(a, Wr[...], preferred_element_type=jnp.float32)
    gs_ref[...] = g * dinvr[...]

  return pl.pallas_call(
      body,
      grid=(N // R,),
      in_specs=[
          pl.BlockSpec((R, D), lambda i: (i, 0)),
          pl.BlockSpec((1, D), lambda i: (0, 0)),
          pl.BlockSpec((1, D), lambda i: (0, 0)),
          pl.BlockSpec((1, D), lambda i: (0, 0)),
          pl.BlockSpec((1, D), lambda i: (0, 0)),
          pl.BlockSpec((D, DO), lambda i: (0, 0)),
          pl.BlockSpec((R, 1), lambda i: (i, 0)),
      ],
      out_specs=pl.BlockSpec((R, DO), lambda i: (i, 0)),
      out_shape=jax.ShapeDtypeStruct((N, DO), jnp.float32),
  )(h, s, q, gamma, beta, W, dinv)


def _tc_agg_bias_stats(a0, a1, gs, dinv, b, R):
  N, D = gs.shape

  def body(a0r, a1r, gsr, dinvr, br, o_ref, s_ref, q_ref):
    op = (a0r[...] + a1r[...] + gsr[...]) * dinvr[...] + br[...]
    o_ref[...] = op

    @pl.when(pl.program_id(0) == 0)
    def _():
      s_ref[...] = jnp.zeros_like(s_ref)
      q_ref[...] = jnp.zeros_like(q_ref)

    s_ref[...] += jnp.sum(op, axis=0, keepdims=True)
    q_ref[...] += jnp.sum(op * op, axis=0, keepdims=True)

  return pl.pallas_call(
      body,
      grid=(N // R,),
      in_specs=[
          pl.BlockSpec((R, D), lambda i: (i, 0)),
          pl.BlockSpec((R, D), lambda i: (i, 0)),
          pl.BlockSpec((R, D), lambda i: (i, 0)),
          pl.BlockSpec((R, 1), lambda i: (i, 0)),
          pl.BlockSpec((1, D), lambda i: (0, 0)),
      ],
      out_specs=[
          pl.BlockSpec((R, D), lambda i: (i, 0)),
          pl.BlockSpec((1, D), lambda i: (0, 0)),
          pl.BlockSpec((1, D), lambda i: (0, 0)),
      ],
      out_shape=[
          jax.ShapeDtypeStruct((N, D), jnp.float32),
          jax.ShapeDtypeStruct((1, D), jnp.float32),
          jax.ShapeDtypeStruct((1, D), jnp.float32),
      ],
  )(a0, a1, gs, dinv, b)


def _tc_bn_apply(op, s, q, gamma, beta, R):
  N, D = op.shape
  inv_n = 1.0 / N

  def body(opr, sr, qr, gr, br, out_ref):
    mean = sr[...] * inv_n
    var = qr[...] * inv_n - mean * mean
    istd = lax.rsqrt(var + 1e-5)
    out_ref[...] = (opr[...] - mean) * istd * gr[...] + br[...]

  return pl.pallas_call(
      body,
      grid=(N // R,),
      in_specs=[
          pl.BlockSpec((R, D), lambda i: (i, 0)),
          pl.BlockSpec((1, D), lambda i: (0, 0)),
          pl.BlockSpec((1, D), lambda i: (0, 0)),
          pl.BlockSpec((1, D), lambda i: (0, 0)),
          pl.BlockSpec((1, D), lambda i: (0, 0)),
      ],
      out_specs=pl.BlockSpec((R, D), lambda i: (i, 0)),
      out_shape=jax.ShapeDtypeStruct((N, D), jnp.float32),
  )(op, s, q, gamma, beta)


def kernel(x, edge_index, W1, b1, gamma1, beta1, W2, b2, gamma2, beta2):
  N, _ = x.shape
  E = edge_index.shape[1]
  R = 2000

  n_acc = ((N + NS * 16 - 1) // (NS * 16)) * (NS * 16)

  src = edge_index[0].astype(jnp.int32)
  dst = edge_index[1].astype(jnp.int32)
  pb = (E + NW * EB - 1) // (NW * EB)
  e_pad = NW * pb * EB
  pad = e_pad - E
  src_p = jnp.concatenate([src, jnp.zeros((pad,), jnp.int32)])
  dst_p = jnp.concatenate([dst, jnp.full((pad,), N, jnp.int32)])
  srcw = src_p.reshape(NW, pb, EB)
  dstw = dst_p.reshape(NW, pb, EB)

  degp = _sc_degree(dstw, n_acc)
  deg0 = degp[0, :N, 0:1]
  deg1 = degp[1, :N, 0:1]

  dinv, xs = _tc_dinv_scale(deg0, deg1, x, R)

  accA = _sc_segment_sum(xs, srcw, dstw, n_acc)
  h, s1, q1 = _tc_agg_matmul_stats(
      accA[0, :N], accA[1, :N], xs, dinv, W1, b1.reshape(1, -1), R)

  gs = _tc_bn_relu_matmul_scale(
      h, s1, q1, gamma1.reshape(1, -1), beta1.reshape(1, -1), W2, dinv, R)

  accB = _sc_segment_sum(gs, srcw, dstw, n_acc)
  op, s2, q2 = _tc_agg_bias_stats(
      accB[0, :N], accB[1, :N], gs, dinv, b2.reshape(1, -1), R)

  return _tc_bn_apply(op, s2, q2, gamma2.reshape(1, -1), beta2.reshape(1, -1), R)


# speedup vs baseline: 2.2131x; 1.0293x over previous
import functools

import jax
import jax.numpy as jnp
from jax import lax
from jax.experimental import pallas as pl
from jax.experimental.pallas import tpu as pltpu
from jax.experimental.pallas import tpu_sc as plsc

NC = 2
NS = 16
NW = NC * NS
EB = 128
DEG_W = 16


def _sc_segment_sum(feat, srcw, dstw, zeros16, n_acc):
  _, D = feat.shape
  _, PB, _ = srcw.shape
  rpt = n_acc // NS
  mesh = plsc.VectorSubcoreMesh(
      core_axis_name="c", subcore_axis_name="s", num_cores=NC, num_subcores=NS)

  @functools.partial(
      pl.kernel,
      out_type=jax.ShapeDtypeStruct((NC * n_acc, D), jnp.float32),
      mesh=mesh,
      scratch_types=[
          pltpu.VMEM((PB, EB), jnp.int32),
          pltpu.VMEM((PB, EB), jnp.int32),
          pltpu.VMEM((EB, D), jnp.float32),
          pltpu.VMEM((16, D), jnp.float32),
          pltpu.VMEM_SHARED((n_acc, D), jnp.float32),
          pltpu.SemaphoreType.DMA,
      ],
  )
  def seg(feat_hbm, src_hbm, dst_hbm, z_hbm, out_hbm,
          srcv, dstv, rowsv, zv, acc, sem):
    c = lax.axis_index("c")
    s = lax.axis_index("s")
    wid = s * NC + c
    pltpu.sync_copy(src_hbm.at[wid], srcv)
    pltpu.sync_copy(dst_hbm.at[wid], dstv)
    pltpu.sync_copy(z_hbm, zv)

    base = s * rpt

    def zblock(i, carry):
      pltpu.sync_copy(zv, acc.at[pl.ds(base + i * 16, 16)])
      return carry
    lax.fori_loop(0, rpt // 16, zblock, 0)
    plsc.subcore_barrier()

    def step(j, carry):
      pltpu.async_copy(feat_hbm.at[srcv.at[j]], rowsv, sem).wait()
      pltpu.sync_copy(rowsv, acc.at[dstv.at[j]], add=True)
      return carry
    lax.fori_loop(0, PB, step, 0)
    plsc.subcore_barrier()

    pltpu.sync_copy(acc.at[pl.ds(base, rpt)],
                    out_hbm.at[pl.ds(c * n_acc + base, rpt)])

  return seg(feat, srcw, dstw, zeros16).reshape(NC, n_acc, D)


def _sc_degree(dstw, ones_rows, zeros16, n_acc):
  _, PB, _ = dstw.shape
  D = DEG_W
  rpt = n_acc // NS
  mesh = plsc.VectorSubcoreMesh(
      core_axis_name="c", subcore_axis_name="s", num_cores=NC, num_subcores=NS)

  @functools.partial(
      pl.kernel,
      out_type=jax.ShapeDtypeStruct((NC * n_acc, D), jnp.float32),
      mesh=mesh,
      scratch_types=[
          pltpu.VMEM((PB, EB), jnp.int32),
          pltpu.VMEM((EB, D), jnp.float32),
          pltpu.VMEM((16, D), jnp.float32),
          pltpu.VMEM_SHARED((n_acc, D), jnp.float32),
      ],
  )
  def deg(dst_hbm, ones_hbm, z_hbm, out_hbm, dstv, onesv, zv, acc):
    c = lax.axis_index("c")
    s = lax.axis_index("s")
    wid = s * NC + c
    pltpu.sync_copy(dst_hbm.at[wid], dstv)
    pltpu.sync_copy(ones_hbm, onesv)
    pltpu.sync_copy(z_hbm, zv)

    base = s * rpt

    def zblock(i, carry):
      pltpu.sync_copy(zv, acc.at[pl.ds(base + i * 16, 16)])
      return carry
    lax.fori_loop(0, rpt // 16, zblock, 0)
    plsc.subcore_barrier()

    def step(j, carry):
      pltpu.sync_copy(onesv, acc.at[dstv.at[j]], add=True)
      return carry
    lax.fori_loop(0, PB, step, 0)
    plsc.subcore_barrier()

    pltpu.sync_copy(acc.at[pl.ds(base, rpt)],
                    out_hbm.at[pl.ds(c * n_acc + base, rpt)])

  return deg(dstw, ones_rows, zeros16).reshape(NC, n_acc, D)


def _tc_dinv_scale(deg0, deg1, x, R):
  N, D = x.shape

  def body(d0, d1, xr, dinv_ref, xs_ref):
    dinv = lax.rsqrt(d0[...] + d1[...] + 1.0)
    dinv_ref[...] = dinv
    xs_ref[...] = xr[...] * dinv

  return pl.pallas_call(
      body,
      grid=(N // R,),
      in_specs=[
          pl.BlockSpec((R, 1), lambda i: (i, 0)),
          pl.BlockSpec((R, 1), lambda i: (i, 0)),
          pl.BlockSpec((R, D), lambda i: (i, 0)),
      ],
      out_specs=[
          pl.BlockSpec((R, 1), lambda i: (i, 0)),
          pl.BlockSpec((R, D), lambda i: (i, 0)),
      ],
      out_shape=[
          jax.ShapeDtypeStruct((N, 1), jnp.float32),
          jax.ShapeDtypeStruct((N, D), jnp.float32),
      ],
  )(deg0, deg1, x)


def _tc_agg_matmul_stats(a0, a1, xs, dinv, W, b, R):
  N, D = xs.shape
  DO = W.shape[1]

  def body(a0r, a1r, xsr, dinvr, Wr, br, h_ref, s_ref, q_ref):
    agg = (a0r[...] + a1r[...] + xsr[...]) * dinvr[...]
    h = jnp.dot(agg, Wr[...], preferred_element_type=jnp.float32) + br[...]
    h_ref[...] = h

    @pl.when(pl.program_id(0) == 0)
    def _():
      s_ref[...] = jnp.zeros_like(s_ref)
      q_ref[...] = jnp.zeros_like(q_ref)

    s_ref[...] += jnp.sum(h, axis=0, keepdims=True)
    q_ref[...] += jnp.sum(h * h, axis=0, keepdims=True)

  return pl.pallas_call(
      body,
      grid=(N // R,),
      in_specs=[
          pl.BlockSpec((R, D), lambda i: (i, 0)),
          pl.BlockSpec((R, D), lambda i: (i, 0)),
          pl.BlockSpec((R, D), lambda i: (i, 0)),
          pl.BlockSpec((R, 1), lambda i: (i, 0)),
          pl.BlockSpec((D, DO), lambda i: (0, 0)),
          pl.BlockSpec((1, DO), lambda i: (0, 0)),
      ],
      out_specs=[
          pl.BlockSpec((R, DO), lambda i: (i, 0)),
          pl.BlockSpec((1, DO), lambda i: (0, 0)),
          pl.BlockSpec((1, DO), lambda i: (0, 0)),
      ],
      out_shape=[
          jax.ShapeDtypeStruct((N, DO), jnp.float32),
          jax.ShapeDtypeStruct((1, DO), jnp.float32),
          jax.ShapeDtypeStruct((1, DO), jnp.float32),
      ],
  )(a0, a1, xs, dinv, W, b)


def _tc_bn_relu_matmul_scale(h, s, q, gamma, beta, W, dinv, R):
  N, D = h.shape
  DO = W.shape[1]
  inv_n = 1.0 / N

  def body(hr, sr, qr, gr, br, Wr, dinvr, gs_ref):
    mean = sr[...] * inv_n
    var = qr[...] * inv_n - mean * mean
    istd = lax.rsqrt(var + 1e-5)
    hn = (hr[...] - mean) * istd * gr[...] + br[...]
    a = jnp.maximum(hn, 0.0)
    g = jnp.dot(a, Wr[...], preferred_element_type=jnp.float32)
    gs_ref[...] = g * dinvr[...]

  return pl.pallas_call(
      body,
      grid=(N // R,),
      in_specs=[
          pl.BlockSpec((R, D), lambda i: (i, 0)),
          pl.BlockSpec((1, D), lambda i: (0, 0)),
          pl.BlockSpec((1, D), lambda i: (0, 0)),
          pl.BlockSpec((1, D), lambda i: (0, 0)),
          pl.BlockSpec((1, D), lambda i: (0, 0)),
          pl.BlockSpec((D, DO), lambda i: (0, 0)),
          pl.BlockSpec((R, 1), lambda i: (i, 0)),
      ],
      out_specs=pl.BlockSpec((R, DO), lambda i: (i, 0)),
      out_shape=jax.ShapeDtypeStruct((N, DO), jnp.float32),
  )(h, s, q, gamma, beta, W, dinv)


def _tc_agg_bias_stats(a0, a1, gs, dinv, b, R):
  N, D = gs.shape

  def body(a0r, a1r, gsr, dinvr, br, o_ref, s_ref, q_ref):
    op = (a0r[...] + a1r[...] + gsr[...]) * dinvr[...] + br[...]
    o_ref[...] = op

    @pl.when(pl.program_id(0) == 0)
    def _():
      s_ref[...] = jnp.zeros_like(s_ref)
      q_ref[...] = jnp.zeros_like(q_ref)

    s_ref[...] += jnp.sum(op, axis=0, keepdims=True)
    q_ref[...] += jnp.sum(op * op, axis=0, keepdims=True)

  return pl.pallas_call(
      body,
      grid=(N // R,),
      in_specs=[
          pl.BlockSpec((R, D), lambda i: (i, 0)),
          pl.BlockSpec((R, D), lambda i: (i, 0)),
          pl.BlockSpec((R, D), lambda i: (i, 0)),
          pl.BlockSpec((R, 1), lambda i: (i, 0)),
          pl.BlockSpec((1, D), lambda i: (0, 0)),
      ],
      out_specs=[
          pl.BlockSpec((R, D), lambda i: (i, 0)),
          pl.BlockSpec((1, D), lambda i: (0, 0)),
          pl.BlockSpec((1, D), lambda i: (0, 0)),
      ],
      out_shape=[
          jax.ShapeDtypeStruct((N, D), jnp.float32),
          jax.ShapeDtypeStruct((1, D), jnp.float32),
          jax.ShapeDtypeStruct((1, D), jnp.float32),
      ],
  )(a0, a1, gs, dinv, b)


def _tc_bn_apply(op, s, q, gamma, beta, R):
  N, D = op.shape
  inv_n = 1.0 / N

  def body(opr, sr, qr, gr, br, out_ref):
    mean = sr[...] * inv_n
    var = qr[...] * inv_n - mean * mean
    istd = lax.rsqrt(var + 1e-5)
    out_ref[...] = (opr[...] - mean) * istd * gr[...] + br[...]

  return pl.pallas_call(
      body,
      grid=(N // R,),
      in_specs=[
          pl.BlockSpec((R, D), lambda i: (i, 0)),
          pl.BlockSpec((1, D), lambda i: (0, 0)),
          pl.BlockSpec((1, D), lambda i: (0, 0)),
          pl.BlockSpec((1, D), lambda i: (0, 0)),
          pl.BlockSpec((1, D), lambda i: (0, 0)),
      ],
      out_specs=pl.BlockSpec((R, D), lambda i: (i, 0)),
      out_shape=jax.ShapeDtypeStruct((N, D), jnp.float32),
  )(op, s, q, gamma, beta)


def kernel(x, edge_index, W1, b1, gamma1, beta1, W2, b2, gamma2, beta2):
  N, _ = x.shape
  E = edge_index.shape[1]
  R = 2000

  n_acc = ((N + NS * 16 - 1) // (NS * 16)) * (NS * 16)

  src = edge_index[0].astype(jnp.int32)
  dst = edge_index[1].astype(jnp.int32)
  pb = (E + NW * EB - 1) // (NW * EB)
  e_pad = NW * pb * EB
  pad = e_pad - E
  src_p = jnp.concatenate([src, jnp.zeros((pad,), jnp.int32)])
  dst_p = jnp.concatenate([dst, jnp.full((pad,), N, jnp.int32)])
  srcw = src_p.reshape(NW, pb, EB)
  dstw = dst_p.reshape(NW, pb, EB)

  degp = _sc_degree(dstw, jnp.ones((EB, DEG_W), jnp.float32),
                    jnp.zeros((16, DEG_W), jnp.float32), n_acc)
  zeros16 = jnp.zeros((16, x.shape[1]), jnp.float32)
  deg0 = degp[0, :N, 0:1]
  deg1 = degp[1, :N, 0:1]

  dinv, xs = _tc_dinv_scale(deg0, deg1, x, R)

  accA = _sc_segment_sum(xs, srcw, dstw, zeros16, n_acc)
  h, s1, q1 = _tc_agg_matmul_stats(
      accA[0, :N], accA[1, :N], xs, dinv, W1, b1.reshape(1, -1), R)

  gs = _tc_bn_relu_matmul_scale(
      h, s1, q1, gamma1.reshape(1, -1), beta1.reshape(1, -1), W2, dinv, R)

  accB = _sc_segment_sum(gs, srcw, dstw, zeros16, n_acc)
  op, s2, q2 = _tc_agg_bias_stats(
      accB[0, :N], accB[1, :N], gs, dinv, b2.reshape(1, -1), R)

  return _tc_bn_apply(op, s2, q2, gamma2.reshape(1, -1), beta2.reshape(1, -1), R)
